# Initial kernel scaffold; baseline (speedup 1.0000x reference)
import jax
import jax.numpy as jnp
from jax.experimental import pallas as pl


def kernel(inputs, alpha, gamma):
    raise NotImplementedError("write your pallas kernel here")


# speedup vs baseline: 70.5437x; 70.5437x over previous
import functools

import jax
import jax.numpy as jnp
from jax import lax
from jax.experimental import pallas as pl
from jax.experimental.pallas import tpu as pltpu
from jax.experimental.pallas import tpu_sc as plsc

B = 128
T = 4096
F = 16
SEASON_LEN = 24
C_OUT = 19
LANES = 16
NUM_CORES = 2
NUM_SUBCORES = 16
NUM_WORKERS = NUM_CORES * NUM_SUBCORES
BATCH_PER_WORKER = B // NUM_WORKERS
NUM_CHUNKS = -(-(T - SEASON_LEN) // LANES)
NSLAB = 4
TS = T // NSLAB


def _pow_iota(base, iota):
    r = jnp.ones((LANES,), jnp.float32)
    b = base
    for bit in range(4):
        m = ((iota >> bit) & 1) == 1
        r = jnp.where(m, r * b, r)
        b = b * b
    return r


def _hw_body(in_hbm, a_hbm, g_hbm, out_hbm, out_v, sin_v, a_v, g_v):
    cid = lax.axis_index("c")
    sid = lax.axis_index("s")
    wid = sid * NUM_CORES + cid

    pltpu.sync_copy(a_hbm, a_v)
    pltpu.sync_copy(g_hbm, g_v)
    av = a_v[...]
    gv = g_v[...]
    oma = 1.0 - av
    omg = 1.0 - gv
    iota = lax.iota(jnp.int32, LANES)
    pw = _pow_iota(oma, iota)
    ipw = _pow_iota(1.0 / oma, iota)

    zeros = jnp.zeros((LANES,), jnp.float32)
    m8 = iota < (SEASON_LEN - LANES)

    for bi in range(BATCH_PER_WORKER):
        b = wid * BATCH_PER_WORKER + bi

        for sl in range(NSLAB):
            pltpu.sync_copy(in_hbm.at[b, pl.ds(sl * TS * F, TS * F)], sin_v)

            @plsc.parallel_loop(0, TS, 1, unroll=8)
            def restripe(t):
                row = sin_v[pl.ds(t * F, LANES)]
                dst = ((sl * TS + t) * C_OUT + 1) + iota
                plsc.store_scatter(out_v, [dst], row)

        f0 = iota * C_OUT
        f1 = (iota + LANES) * C_OUT
        p0 = plsc.load_gather(out_v, [f0 + 1])
        p1 = plsc.load_gather(out_v, [f1 + 1])
        init = (jnp.sum(p0) + jnp.sum(jnp.where(m8, p1, 0.0))) * (
            1.0 / SEASON_LEN)
        init_v = lax.broadcast(init, (LANES,))
        plsc.store_scatter(out_v, [f0], p0)
        plsc.store_scatter(out_v, [f0 + (C_OUT - 2)], init_v)
        plsc.store_scatter(out_v, [f0 + (C_OUT - 1)], zeros)
        plsc.store_scatter(out_v, [f1], p1, mask=m8)
        plsc.store_scatter(out_v, [f1 + (C_OUT - 2)], init_v, mask=m8)
        plsc.store_scatter(out_v, [f1 + (C_OUT - 1)], zeros, mask=m8)

        def chunk(i, lprev):
            t0 = SEASON_LEN + LANES * i
            rows_raw = t0 + iota
            valid = rows_raw < T
            rows = jnp.minimum(rows_raw, T - 1)
            fr = rows * C_OUT
            p = plsc.load_gather(out_v, [fr + 1])
            slag = plsc.load_gather(
                out_v, [fr - (SEASON_LEN * C_OUT - (C_OUT - 1))])
            w = av * (p - slag) * ipw
            cs = plsc.cumsum(w)
            l = pw * (oma * lprev + cs)
            s = gv * (p - l) + omg * slag
            y = p - s
            plsc.store_scatter(out_v, [fr], y, mask=valid)
            plsc.store_scatter(out_v, [fr + (C_OUT - 2)], l, mask=valid)
            plsc.store_scatter(out_v, [fr + (C_OUT - 1)], s, mask=valid)
            return jnp.sum(jnp.where(iota == LANES - 1, l, 0.0))

        lax.fori_loop(0, NUM_CHUNKS, chunk, init)

        pltpu.sync_copy(out_v, out_hbm.at[b])


def kernel(inputs, alpha, gamma):
    mesh = plsc.VectorSubcoreMesh(
        core_axis_name="c", subcore_axis_name="s",
        num_cores=NUM_CORES, num_subcores=NUM_SUBCORES)
    hw = functools.partial(
        pl.kernel,
        out_type=jax.ShapeDtypeStruct((B, T * C_OUT), jnp.float32),
        mesh=mesh,
        scratch_types=[
            pltpu.VMEM((T * C_OUT,), jnp.float32),
            pltpu.VMEM((TS * F,), jnp.float32),
            pltpu.VMEM((LANES,), jnp.float32),
            pltpu.VMEM((LANES,), jnp.float32),
        ],
        compiler_params=pltpu.CompilerParams(
            needs_layout_passes=False, use_tc_tiling_on_sc=False),
    )(_hw_body)
    a16 = jnp.broadcast_to(alpha.astype(jnp.float32), (LANES,))
    g16 = jnp.broadcast_to(gamma.astype(jnp.float32), (LANES,))
    out2 = hw(inputs.reshape(B, T * F), a16, g16)
    return out2.reshape(B, T, C_OUT)
